# Initial kernel scaffold; baseline (speedup 1.0000x reference)
import jax
import jax.numpy as jnp
from jax.experimental import pallas as pl


def kernel(pts, timestamps, aabb, table0, table1, table2, table3, table4, table5):
    raise NotImplementedError("write your pallas kernel here")


# speedup vs baseline: 3.3927x; 3.3927x over previous
import functools

import jax
import jax.numpy as jnp
import numpy as np
from jax import lax
from jax.experimental import pallas as pl
from jax.experimental.pallas import tpu as pltpu
from jax.experimental.pallas import tpu_sc as plsc

N_PTS = 131072
N_LEVELS = 16
F_PER_LEVEL = 8
NFEAT = N_LEVELS * F_PER_LEVEL
LOG2_T = 16
TABLE_SIZE = 1 << LOG2_T
BASE_RES = 32
PRIME2_I32 = np.int32(2654435761 - (1 << 32))

NC, NS, LANES = 2, 16, 16
NW = NC * NS
PTS_PER_W = N_PTS // NW
G = 128
N_GROUPS = PTS_PER_W // G
SG = G // LANES


def _floor_parts(pos):
    pi0 = pos.astype(jnp.int32)
    pif = pi0.astype(jnp.float32)
    neg = pos < pif
    pi = jnp.where(neg, pi0 - 1, pi0)
    pf = jnp.where(neg, pif - 1.0, pif)
    return pi, pos - pf


def _sc_body(coords, t0, t1, t2, t3, t4, t5, out,
             cbuf, ibuf, wbuf, gbuf, obuf, sem):
    tbls = [t0, t1, t2, t3, t4, t5]
    wid = lax.axis_index("s") * NC + lax.axis_index("c")
    iota = lax.iota(jnp.int32, LANES)
    iota8 = iota * F_PER_LEVEL
    iota128 = iota * NFEAT

    def group_body(g, carry):
        base = wid * PTS_PER_W + g * G
        base = pl.multiple_of(base, G)
        pltpu.sync_copy(coords.at[:, :, pl.ds(base, G)], cbuf)

        def level_body(l, carry2):
            scale = ((jnp.int32(BASE_RES) << l) - 1).astype(jnp.float32)
            lvl_off = l << LOG2_T

            def hash_body(s, c3):
                off = pl.multiple_of(s * LANES, LANES)
                for p6 in range(6):
                    xa = cbuf[p6, 0, pl.ds(off, LANES)]
                    xb = cbuf[p6, 1, pl.ds(off, LANES)]
                    posx = xa * scale + 0.5
                    posy = xb * scale + 0.5
                    pix, wx = _floor_parts(posx)
                    piy, wy = _floor_parts(posy)
                    a0 = pix
                    a1 = pix + 1
                    b0 = piy * PRIME2_I32
                    b1 = b0 + PRIME2_I32
                    for c, h in enumerate((a0 ^ b0, a1 ^ b0, a0 ^ b1, a1 ^ b1)):
                        idx = (h & jnp.int32(TABLE_SIZE - 1)) + lvl_off
                        ibuf[p6 * 4 + c, pl.ds(off, LANES)] = idx
                    wbuf[p6 * 2, pl.ds(off, LANES)] = wx
                    wbuf[p6 * 2 + 1, pl.ds(off, LANES)] = wy
                return c3

            lax.fori_loop(0, SG, hash_body, 0)

            cps = []
            for j in range(24):
                p6 = j // 4
                cps.append(pltpu.async_copy(
                    tbls[p6].at[ibuf.at[j]], gbuf.at[pl.ds(j * G, G)], sem))
            for cp in cps:
                cp.wait()

            def mac_body(s, c3):
                soff = pl.multiple_of(s * LANES, LANES)
                acc = [None] * F_PER_LEVEL
                for p6 in range(6):
                    wx = wbuf[p6 * 2, pl.ds(soff, LANES)]
                    wy = wbuf[p6 * 2 + 1, pl.ds(soff, LANES)]
                    rows = [iota + (p6 * 4 + c) * G + s * LANES
                            for c in range(4)]
                    for f in range(F_PER_LEVEL):
                        col = jnp.full((LANES,), f, jnp.int32)
                        r00 = plsc.load_gather(gbuf, [rows[0], col])
                        r10 = plsc.load_gather(gbuf, [rows[1], col])
                        r01 = plsc.load_gather(gbuf, [rows[2], col])
                        r11 = plsc.load_gather(gbuf, [rows[3], col])
                        r0 = r00 + wx * (r10 - r00)
                        r1 = r01 + wx * (r11 - r01)
                        v = r0 + wy * (r1 - r0)
                        acc[f] = v if p6 == 0 else acc[f] * v
                for f in range(F_PER_LEVEL):
                    plsc.store_scatter(
                        obuf, [iota128 + (s * LANES * NFEAT + l * F_PER_LEVEL + f)],
                        acc[f])
                return c3

            lax.fori_loop(0, SG, mac_body, 0)
            return carry2

        lax.fori_loop(0, N_LEVELS, level_body, 0)
        pltpu.sync_copy(obuf, out.at[pl.ds(base * NFEAT, G * NFEAT)])
        return carry

    lax.fori_loop(0, N_GROUPS, group_body, 0)


@jax.jit
def _encode(coords, t0, t1, t2, t3, t4, t5):
    mesh = plsc.VectorSubcoreMesh(core_axis_name="c", subcore_axis_name="s")
    fn = functools.partial(
        pl.kernel,
        mesh=mesh,
        out_type=jax.ShapeDtypeStruct((N_PTS * NFEAT,), jnp.float32),
        scratch_types=[
            pltpu.VMEM((6, 2, G), jnp.float32),
            pltpu.VMEM((24, G), jnp.int32),
            pltpu.VMEM((12, G), jnp.float32),
            pltpu.VMEM((24 * G, F_PER_LEVEL), jnp.float32),
            pltpu.VMEM((G * NFEAT,), jnp.float32),
            pltpu.SemaphoreType.DMA,
        ],
        compiler_params=pltpu.CompilerParams(
            use_tc_tiling_on_sc=False, needs_layout_passes=False),
    )(_sc_body)
    return fn(coords, t0, t1, t2, t3, t4, t5)


def kernel(pts, timestamps, aabb, table0, table1, table2, table3, table4, table5):
    pts_n = (pts - aabb[0]) * (2.0 / (aabb[1] - aabb[0])) - 1.0
    p4 = jnp.concatenate([pts_n, timestamps], axis=-1)
    combos = [(0, 1), (0, 2), (0, 3), (1, 2), (1, 3), (2, 3)]
    coords = jnp.stack([p4[:, (a, b)].T for a, b in combos])
    flat = _encode(coords, table0, table1, table2, table3, table4, table5)
    return flat.reshape(N_PTS, NFEAT)


# speedup vs baseline: 3.4093x; 1.0049x over previous
import functools

import jax
import jax.numpy as jnp
import numpy as np
from jax import lax
from jax.experimental import pallas as pl
from jax.experimental.pallas import tpu as pltpu
from jax.experimental.pallas import tpu_sc as plsc

N_PTS = 131072
N_LEVELS = 16
F_PER_LEVEL = 8
NFEAT = N_LEVELS * F_PER_LEVEL
LOG2_T = 16
TABLE_SIZE = 1 << LOG2_T
BASE_RES = 32
PRIME2_I32 = np.int32(2654435761 - (1 << 32))

NC, NS, LANES = 2, 16, 16
NW = NC * NS
PTS_PER_W = N_PTS // NW
G = 128
N_GROUPS = PTS_PER_W // G
SG = G // LANES


def _floor_parts(pos):
    pi0 = pos.astype(jnp.int32)
    pif = pi0.astype(jnp.float32)
    neg = pos < pif
    pi = jnp.where(neg, pi0 - 1, pi0)
    pf = jnp.where(neg, pif - 1.0, pif)
    return pi, pos - pf


COMBOS = ((0, 1), (0, 2), (0, 3), (1, 2), (1, 3), (2, 3))


def _sc_body(coords, t0, t1, t2, t3, t4, t5, out,
             cbuf, ibuf, wbuf, gbuf, obuf, sem):
    tbls = [t0, t1, t2, t3, t4, t5]
    wid = lax.axis_index("s") * NC + lax.axis_index("c")
    iota = lax.iota(jnp.int32, LANES)

    def group_body(g, carry):
        base = wid * PTS_PER_W + g * G
        base = pl.multiple_of(base, G)
        pltpu.sync_copy(coords.at[:, pl.ds(base, G)], cbuf)

        def level_body(l, carry2):
            scale = ((jnp.int32(BASE_RES) << l) - 1).astype(jnp.float32)
            lvl_off = l << LOG2_T

            def hash_body(s, c3):
                off = pl.multiple_of(s * LANES, LANES)
                for p6, (ca, cb2) in enumerate(COMBOS):
                    xa = cbuf[ca, pl.ds(off, LANES)]
                    xb = cbuf[cb2, pl.ds(off, LANES)]
                    posx = xa * scale + 0.5
                    posy = xb * scale + 0.5
                    pix, wx = _floor_parts(posx)
                    piy, wy = _floor_parts(posy)
                    a0 = pix
                    a1 = pix + 1
                    b0 = piy * PRIME2_I32
                    b1 = b0 + PRIME2_I32
                    for c, h in enumerate((a0 ^ b0, a1 ^ b0, a0 ^ b1, a1 ^ b1)):
                        idx = (h & jnp.int32(TABLE_SIZE - 1)) + lvl_off
                        ibuf[p6 * 4 + c, pl.ds(off, LANES)] = idx
                    wbuf[p6 * 2, pl.ds(off, LANES)] = wx
                    wbuf[p6 * 2 + 1, pl.ds(off, LANES)] = wy
                return c3

            lax.fori_loop(0, SG, hash_body, 0)

            cps = []
            for j in range(24):
                p6 = j // 4
                cps.append(pltpu.async_copy(
                    tbls[p6].at[ibuf.at[j]], gbuf.at[pl.ds(j * G, G)], sem))
            for cp in cps:
                cp.wait()

            def mac_body(s, c3):
                soff = pl.multiple_of(s * LANES, LANES)
                acc = [None] * F_PER_LEVEL
                for p6 in range(6):
                    wx = wbuf[p6 * 2, pl.ds(soff, LANES)]
                    wy = wbuf[p6 * 2 + 1, pl.ds(soff, LANES)]
                    rows = [iota + (p6 * 4 + c) * G + s * LANES
                            for c in range(4)]
                    for f in range(F_PER_LEVEL):
                        col = jnp.full((LANES,), f, jnp.int32)
                        r00 = plsc.load_gather(gbuf, [rows[0], col])
                        r10 = plsc.load_gather(gbuf, [rows[1], col])
                        r01 = plsc.load_gather(gbuf, [rows[2], col])
                        r11 = plsc.load_gather(gbuf, [rows[3], col])
                        r0 = r00 + wx * (r10 - r00)
                        r1 = r01 + wx * (r11 - r01)
                        v = r0 + wy * (r1 - r0)
                        acc[f] = v if p6 == 0 else acc[f] * v
                for f in range(F_PER_LEVEL):
                    plsc.store_scatter(
                        obuf, [iota + soff,
                               jnp.full((LANES,), f, jnp.int32) + l * F_PER_LEVEL],
                        acc[f])
                return c3

            lax.fori_loop(0, SG, mac_body, 0)
            return carry2

        lax.fori_loop(0, N_LEVELS, level_body, 0)
        pltpu.sync_copy(obuf, out.at[pl.ds(base, G)])
        return carry

    lax.fori_loop(0, N_GROUPS, group_body, 0)


@jax.jit
def _encode(coords, t0, t1, t2, t3, t4, t5):
    mesh = plsc.VectorSubcoreMesh(core_axis_name="c", subcore_axis_name="s")
    fn = functools.partial(
        pl.kernel,
        mesh=mesh,
        out_type=jax.ShapeDtypeStruct((N_PTS, NFEAT), jnp.float32),
        scratch_types=[
            pltpu.VMEM((4, G), jnp.float32),
            pltpu.VMEM((24, G), jnp.int32),
            pltpu.VMEM((12, G), jnp.float32),
            pltpu.VMEM((24 * G, F_PER_LEVEL), jnp.float32),
            pltpu.VMEM((G, NFEAT), jnp.float32),
            pltpu.SemaphoreType.DMA,
        ],
        compiler_params=pltpu.CompilerParams(
            use_tc_tiling_on_sc=False, needs_layout_passes=False),
    )(_sc_body)
    return fn(coords, t0, t1, t2, t3, t4, t5)


def kernel(pts, timestamps, aabb, table0, table1, table2, table3, table4, table5):
    pts_n = (pts - aabb[0]) * (2.0 / (aabb[1] - aabb[0])) - 1.0
    p4 = jnp.concatenate([pts_n, timestamps], axis=-1)
    coords = p4.T
    return _encode(coords, table0, table1, table2, table3, table4, table5)


# speedup vs baseline: 4.6588x; 1.3665x over previous
import functools

import jax
import jax.numpy as jnp
import numpy as np
from jax import lax
from jax.experimental import pallas as pl
from jax.experimental.pallas import tpu as pltpu
from jax.experimental.pallas import tpu_sc as plsc

N_PTS = 131072
N_LEVELS = 16
F_PER_LEVEL = 8
NFEAT = N_LEVELS * F_PER_LEVEL
LOG2_T = 16
TABLE_SIZE = 1 << LOG2_T
BASE_RES = 32
PRIME2_I32 = np.int32(2654435761 - (1 << 32))

NC, NS, LANES = 2, 16, 16
NW = NC * NS
PTS_PER_W = N_PTS // NW
G = 128
N_GROUPS = PTS_PER_W // G
SG = G // LANES

COMBOS = ((0, 1), (0, 2), (0, 3), (1, 2), (1, 3), (2, 3))


def _floor_parts(pos):
    pi0 = pos.astype(jnp.int32)
    pif = pi0.astype(jnp.float32)
    neg = pos < pif
    pi = jnp.where(neg, pi0 - 1, pi0)
    pf = jnp.where(neg, pif - 1.0, pif)
    return pi, pos - pf


def _sc_body(coords, t0, t1, t2, t3, t4, t5, out,
             cbuf, ibuf0, ibuf1, wbuf0, wbuf1, gbuf0, gbuf1, obuf,
             sem0, sem1):
    tbls = [t0, t1, t2, t3, t4, t5]
    wid = lax.axis_index("s") * NC + lax.axis_index("c")
    iota = lax.iota(jnp.int32, LANES)

    def hash_stage(l, ibuf, wbuf):
        scale = ((jnp.int32(BASE_RES) << l) - 1).astype(jnp.float32)
        lvl_off = l << LOG2_T

        def body(s, c3):
            off = pl.multiple_of(s * LANES, LANES)
            for p6, (ca, cb2) in enumerate(COMBOS):
                xa = cbuf[ca, pl.ds(off, LANES)]
                xb = cbuf[cb2, pl.ds(off, LANES)]
                posx = xa * scale + 0.5
                posy = xb * scale + 0.5
                pix, wx = _floor_parts(posx)
                piy, wy = _floor_parts(posy)
                a0 = pix
                a1 = pix + 1
                b0 = piy * PRIME2_I32
                b1 = b0 + PRIME2_I32
                for c, h in enumerate((a0 ^ b0, a1 ^ b0, a0 ^ b1, a1 ^ b1)):
                    idx = (h & jnp.int32(TABLE_SIZE - 1)) + lvl_off
                    ibuf[p6 * 4 + c, pl.ds(off, LANES)] = idx
                wbuf[p6 * 2, pl.ds(off, LANES)] = wx
                wbuf[p6 * 2 + 1, pl.ds(off, LANES)] = wy
            return c3

        lax.fori_loop(0, SG, body, 0)

    def copies(ibuf, gbuf, sem):
        return [pltpu.make_async_copy(
            tbls[j // 4].at[ibuf.at[j]], gbuf.at[pl.ds(j * G, G)], sem)
            for j in range(24)]

    def fire(ibuf, gbuf, sem):
        for cp in copies(ibuf, gbuf, sem):
            cp.start()

    def drain(ibuf, gbuf, sem):
        for cp in copies(ibuf, gbuf, sem):
            cp.wait()

    def mac_stage(l, wbuf, gbuf):
        def body(s, c3):
            soff = pl.multiple_of(s * LANES, LANES)
            acc = [None] * F_PER_LEVEL
            for p6 in range(6):
                wx = wbuf[p6 * 2, pl.ds(soff, LANES)]
                wy = wbuf[p6 * 2 + 1, pl.ds(soff, LANES)]
                rows = [iota + (p6 * 4 + c) * G + soff for c in range(4)]
                for f in range(F_PER_LEVEL):
                    col = jnp.full((LANES,), f, jnp.int32)
                    r00 = plsc.load_gather(gbuf, [rows[0], col])
                    r10 = plsc.load_gather(gbuf, [rows[1], col])
                    r01 = plsc.load_gather(gbuf, [rows[2], col])
                    r11 = plsc.load_gather(gbuf, [rows[3], col])
                    r0 = r00 + wx * (r10 - r00)
                    r1 = r01 + wx * (r11 - r01)
                    v = r0 + wy * (r1 - r0)
                    acc[f] = v if p6 == 0 else acc[f] * v
            for f in range(F_PER_LEVEL):
                plsc.store_scatter(
                    obuf, [iota + soff,
                           jnp.full((LANES,), f, jnp.int32) + l * F_PER_LEVEL],
                    acc[f])
            return c3

        lax.fori_loop(0, SG, body, 0)

    def group_body(g, carry):
        base = wid * PTS_PER_W + g * G
        base = pl.multiple_of(base, G)
        pltpu.sync_copy(coords.at[:, pl.ds(base, G)], cbuf)

        hash_stage(0, ibuf0, wbuf0)
        fire(ibuf0, gbuf0, sem0)

        def it_body(it, carry2):
            la = 2 * it
            hash_stage(la + 1, ibuf1, wbuf1)
            fire(ibuf1, gbuf1, sem1)
            drain(ibuf0, gbuf0, sem0)
            mac_stage(la, wbuf0, gbuf0)

            @pl.when(it < N_LEVELS // 2 - 1)
            def _():
                hash_stage(la + 2, ibuf0, wbuf0)
                fire(ibuf0, gbuf0, sem0)

            drain(ibuf1, gbuf1, sem1)
            mac_stage(la + 1, wbuf1, gbuf1)
            return carry2

        lax.fori_loop(0, N_LEVELS // 2, it_body, 0)
        pltpu.sync_copy(obuf, out.at[pl.ds(base, G)])
        return carry

    lax.fori_loop(0, N_GROUPS, group_body, 0)


@jax.jit
def _encode(coords, t0, t1, t2, t3, t4, t5):
    mesh = plsc.VectorSubcoreMesh(core_axis_name="c", subcore_axis_name="s")
    fn = functools.partial(
        pl.kernel,
        mesh=mesh,
        out_type=jax.ShapeDtypeStruct((N_PTS, NFEAT), jnp.float32),
        scratch_types=[
            pltpu.VMEM((4, G), jnp.float32),
            pltpu.VMEM((24, G), jnp.int32),
            pltpu.VMEM((24, G), jnp.int32),
            pltpu.VMEM((12, G), jnp.float32),
            pltpu.VMEM((12, G), jnp.float32),
            pltpu.VMEM((24 * G, F_PER_LEVEL), jnp.float32),
            pltpu.VMEM((24 * G, F_PER_LEVEL), jnp.float32),
            pltpu.VMEM((G, NFEAT), jnp.float32),
            pltpu.SemaphoreType.DMA,
            pltpu.SemaphoreType.DMA,
        ],
        compiler_params=pltpu.CompilerParams(
            use_tc_tiling_on_sc=False, needs_layout_passes=False),
    )(_sc_body)
    return fn(coords, t0, t1, t2, t3, t4, t5)


def kernel(pts, timestamps, aabb, table0, table1, table2, table3, table4, table5):
    pts_n = (pts - aabb[0]) * (2.0 / (aabb[1] - aabb[0])) - 1.0
    p4 = jnp.concatenate([pts_n, timestamps], axis=-1)
    coords = p4.T
    return _encode(coords, table0, table1, table2, table3, table4, table5)
